# baseline (speedup 1.0000x reference)
import functools

import jax
import jax.numpy as jnp
from jax import lax
from jax.experimental import pallas as pl
from jax.experimental.pallas import tpu as pltpu
from jax.experimental.pallas import tpu_sc as plsc

BATCH = 4096
HIST_LEN = 200
HIDDEN = 128
CHUNK = 128

_NC = 2
_NS = 16
_NW = _NC * _NS

_N_ROWS = BATCH * HIST_LEN
_ROWS_PER_W = _N_ROWS // _NW
_CHUNKS_PER_W = _ROWS_PER_W // CHUNK
_NBUF = 6
_DEPTH = 4
_GROUPS = _CHUNKS_PER_W // _NBUF


def _make_gather():
    mesh = plsc.VectorSubcoreMesh(core_axis_name="c", subcore_axis_name="s")

    @functools.partial(
        pl.kernel,
        mesh=mesh,
        out_type=jax.ShapeDtypeStruct((_N_ROWS, HIDDEN), jnp.float32),
        scratch_types=[
            pltpu.VMEM((_CHUNKS_PER_W, CHUNK), jnp.int32),
            pltpu.VMEM((_NBUF, CHUNK, HIDDEN), jnp.float32),
        ]
        + [pltpu.SemaphoreType.DMA] * (2 * _NBUF),
    )
    def grab(idx_hbm, table_hbm, out_hbm, idx_v, bufs, *sems):
        sg, sw = sems[:_NBUF], sems[_NBUF:]
        wid = lax.axis_index("s") * _NC + lax.axis_index("c")
        base_chunk = wid * _CHUNKS_PER_W
        pltpu.sync_copy(idx_hbm.at[pl.ds(base_chunk, _CHUNKS_PER_W)], idx_v)

        def fire_gather(j, b):
            pltpu.async_copy(table_hbm.at[idx_v.at[j]], bufs.at[b], sg[b])

        def wait_gather(b):
            pltpu.make_async_copy(
                table_hbm.at[idx_v.at[0]], bufs.at[b], sg[b]
            ).wait()

        def fire_write(j, b):
            pltpu.async_copy(
                bufs.at[b], out_hbm.at[pl.ds((base_chunk + j) * CHUNK, CHUNK)], sw[b]
            )

        def wait_write(b):
            pltpu.make_async_copy(
                bufs.at[b], out_hbm.at[pl.ds(0, CHUNK)], sw[b]
            ).wait()

        for b in range(_DEPTH):
            fire_gather(b, b)

        def body(it, carry):
            ja = it * _NBUF
            for s in range(_NBUF):
                j = ja + s
                jf = j + _DEPTH
                bf = (s + _DEPTH) % _NBUF

                @pl.when(jf < _CHUNKS_PER_W)
                def _(jf=jf, bf=bf):
                    @pl.when(jf >= _NBUF)
                    def _():
                        wait_write(bf)

                    fire_gather(jf, bf)

                wait_gather(s)
                fire_write(j, s)
            return carry

        lax.fori_loop(0, _GROUPS, body, 0)

        for j in range(_GROUPS * _NBUF, _CHUNKS_PER_W):
            wait_gather(j % _NBUF)
            fire_write(j, j % _NBUF)
        for b in range(_NBUF):
            wait_write(b)

    return grab


_gather = _make_gather()


def kernel(input, weight):
    idx = input.reshape(_N_ROWS // CHUNK, CHUNK).astype(jnp.int32)
    out = _gather(idx, weight)
    return out.reshape(BATCH, HIST_LEN, HIDDEN)
